# baseline (speedup 1.0000x reference)
import functools

import jax
import jax.numpy as jnp
from jax import lax
from jax.experimental import pallas as pl
from jax.experimental.pallas import tpu as pltpu
from jax.experimental.pallas import tpu_sc as plsc

N = 10000
E = 20000
G = 256
H = 64
EH = 32
STEPS = 3
C = 100

NP = 10240
EP = 20480
NW = 32
EPW = EP // NW
NCH = EPW // 128
NPW = NP // NW
NPT = NP // 16
HP = 128

F32 = jnp.float32
BF16 = jnp.bfloat16
NEG_INF = float("-inf")


def _proj_body(x_ref, w_ref, b_ref, o_ref, *, nvalid, bm):
    i = pl.program_id(0)
    y = jnp.dot(x_ref[...].astype(BF16), w_ref[...],
                preferred_element_type=F32) + b_ref[...]
    y = jnp.maximum(y, 0.0)
    rows = i * bm + lax.broadcasted_iota(jnp.int32, (bm, 1), 0)
    o_ref[...] = jnp.where(rows < nvalid, y, 0.0)


def _proj(x, w, b, nvalid, mout, bm=2048):
    k = x.shape[1]
    n = w.shape[1]
    return pl.pallas_call(
        functools.partial(_proj_body, nvalid=nvalid, bm=bm),
        grid=(mout // bm,),
        in_specs=[
            pl.BlockSpec((bm, k), lambda i: (i, 0)),
            pl.BlockSpec((k, n), lambda i: (0, 0)),
            pl.BlockSpec((1, n), lambda i: (0, 0)),
        ],
        out_specs=pl.BlockSpec((bm, n), lambda i: (i, 0)),
        out_shape=jax.ShapeDtypeStruct((mout, n), F32),
    )(x, w, b)


def _gather_body(tbl_hbm, idx_hbm, out_hbm, idx_v, rows_v, sem, *, nch):
    c = lax.axis_index("c")
    s = lax.axis_index("s")
    wid = s * 2 + c
    epw = nch * 128
    pltpu.sync_copy(idx_hbm.at[wid], idx_v)
    copies = []
    for j in range(nch):
        copies.append(
            pltpu.async_copy(tbl_hbm.at[idx_v.at[j]],
                             rows_v.at[pl.ds(j * 128, 128)], sem))
    for cp in copies:
        cp.wait()
    pltpu.sync_copy(rows_v, out_hbm.at[pl.ds(wid * epw, epw)])


def _sc_gather(table, idx, nch):
    mesh = plsc.VectorSubcoreMesh(core_axis_name="c", subcore_axis_name="s")
    return pl.kernel(
        functools.partial(_gather_body, nch=nch),
        out_type=jax.ShapeDtypeStruct((NW * nch * 128, HP), F32),
        mesh=mesh,
        scratch_types=[
            pltpu.VMEM((nch, 128), jnp.int32),
            pltpu.VMEM((nch * 128, HP), F32),
            pltpu.SemaphoreType.DMA,
        ],
    )(table, idx)


def _msg_body(g_ref, z_ref, w2_ref, s_ref, b2_ref, o_ref, *, bm, evalid):
    i = pl.program_id(0)
    gb = g_ref[...].astype(BF16)
    zb = z_ref[...].astype(BF16)
    ck = 256
    acc256 = jnp.zeros((bm, ck), F32)
    for c in range(EH * H // ck):
        sl = pl.ds(c * ck, ck)
        yc = jnp.dot(gb, w2_ref[:, sl], preferred_element_type=F32)
        zc = jnp.dot(zb, s_ref[:, sl], preferred_element_type=F32)
        acc256 = acc256 + yc * zc
    acc128 = acc256[:, :128] + acc256[:, 128:]
    m = acc128[:, :H] + acc128[:, H:] + jnp.dot(gb, b2_ref[...],
                                                preferred_element_type=F32)
    m128 = jnp.concatenate([m, jnp.zeros((bm, HP - H), F32)], axis=1)
    if evalid is not None:
        rows = i * bm + lax.broadcasted_iota(jnp.int32, (bm, 1), 0)
        m128 = jnp.where(rows < evalid, m128, 0.0)
    o_ref[...] = m128


def _messages(g, z, w2cat, smat, b2, evalid, bm=512):
    ep = g.shape[0]
    return pl.pallas_call(
        functools.partial(_msg_body, bm=bm, evalid=evalid),
        grid=(ep // bm,),
        in_specs=[
            pl.BlockSpec((bm, HP), lambda i: (i, 0)),
            pl.BlockSpec((bm, EH), lambda i: (i, 0)),
            pl.BlockSpec((HP, EH * H), lambda i: (0, 0)),
            pl.BlockSpec((EH, EH * H), lambda i: (0, 0)),
            pl.BlockSpec((HP, H), lambda i: (0, 0)),
        ],
        out_specs=pl.BlockSpec((bm, HP), lambda i: (i, 0)),
        out_shape=jax.ShapeDtypeStruct((ep, HP), F32),
    )(g, z, w2cat, smat, b2)


def _scatter_body(ma_hbm, mb_hbm, idxa_hbm, idxb_hbm, zero_hbm, part_hbm,
                  m_v0, m_v1, idxa_v, idxb_v, acc_sh, sem0, sem1, *,
                  nch_a, nch_b):
    c = lax.axis_index("c")
    s = lax.axis_index("s")
    wid = s * 2 + c
    bufs = (m_v0, m_v1)
    sems = (sem0, sem1)
    def chunk_src(j):
        if j < nch_a:
            return ma_hbm.at[pl.ds(wid * nch_a * 128 + j * 128, 128)]
        return mb_hbm.at[pl.ds(wid * nch_b * 128 + (j - nch_a) * 128, 128)]
    def chunk_idx(j):
        if j < nch_a:
            return idxa_v.at[j]
        return idxb_v.at[j - nch_a]
    nch = nch_a + nch_b
    pltpu.sync_copy(zero_hbm, m_v0)
    for j in range(NPT // 128):
        pltpu.sync_copy(m_v0, acc_sh.at[pl.ds(s * NPT + j * 128, 128)])
    pltpu.sync_copy(idxa_hbm.at[wid], idxa_v)
    pltpu.sync_copy(idxb_hbm.at[wid], idxb_v)
    cps = [None, None]
    cps[0] = pltpu.async_copy(chunk_src(0), m_v0, sem0)
    plsc.subcore_barrier()
    for j in range(nch):
        if j + 1 < nch:
            cps[(j + 1) % 2] = pltpu.async_copy(
                chunk_src(j + 1), bufs[(j + 1) % 2], sems[(j + 1) % 2])
        cps[j % 2].wait()
        pltpu.sync_copy(bufs[j % 2], acc_sh.at[chunk_idx(j)], add=True)
    plsc.subcore_barrier()
    pltpu.sync_copy(acc_sh.at[pl.ds(s * NPT, NPT)],
                    part_hbm.at[c, pl.ds(s * NPT, NPT)])


def _sc_scatter(ma, mb, idxa, idxb, zeros_tile, nch_a, nch_b):
    mesh = plsc.VectorSubcoreMesh(core_axis_name="c", subcore_axis_name="s")
    return pl.kernel(
        functools.partial(_scatter_body, nch_a=nch_a, nch_b=nch_b),
        out_type=jax.ShapeDtypeStruct((2, NP, HP), F32),
        mesh=mesh,
        scratch_types=[
            pltpu.VMEM((128, HP), F32),
            pltpu.VMEM((128, HP), F32),
            pltpu.VMEM((nch_a, 128), jnp.int32),
            pltpu.VMEM((nch_b, 128), jnp.int32),
            pltpu.VMEM_SHARED((NP, HP), F32),
            pltpu.SemaphoreType.DMA,
            pltpu.SemaphoreType.DMA,
        ],
    )(ma, mb, idxa, idxb, zeros_tile)


def _gru_body(p_ref, hid_ref, wih_ref, whh_ref, bih_ref, bhh_ref, bc_ref,
              ww_ref, bw_ref, h_ref, *extra_refs, final, bm):
    i = pl.program_id(0)
    hid = hid_ref[:, :H]
    x = jnp.maximum(p_ref[0] + p_ref[1] + bc_ref[...], 0.0)
    gs = (jnp.dot(x.astype(BF16), wih_ref[...], preferred_element_type=F32)
          + jnp.dot(hid_ref[...].astype(BF16), whh_ref[...],
                    preferred_element_type=F32) + bih_ref[...])
    gh_n = (jnp.dot(hid_ref[...].astype(BF16), whh_ref[:, 2 * 128:2 * 128 + H],
                    preferred_element_type=F32) + bhh_ref[...])
    r = jax.nn.sigmoid(gs[:, 0:H])
    zg = jax.nn.sigmoid(gs[:, 128:128 + H])
    n = jnp.tanh(gs[:, 256:256 + H] - gh_n + r * gh_n)
    hn = (1.0 - zg) * n + zg * hid
    rows = i * bm + lax.broadcasted_iota(jnp.int32, (bm, 1), 0)
    valid = rows < N
    hn = jnp.where(valid, hn, 0.0)
    h_ref[...] = jnp.concatenate([hn, jnp.zeros((bm, HP - H), F32)], axis=1)
    if final:
        hw_ref, hm_ref = extra_refs
        wv = jax.nn.sigmoid(jnp.dot(hn.astype(BF16), ww_ref[...],
                                    preferred_element_type=F32) + bw_ref[...])
        hw_ref[...] = jnp.where(valid, hn * wv, 0.0)
        hm_ref[...] = jnp.where(valid, hn, NEG_INF)


def _gru(parts, hid, wih, whh, bih, bhhn, bc, ww, bw, final, bm=1024):
    nout = 3 if final else 1
    shapes = [jax.ShapeDtypeStruct((NP, HP), F32)]
    specs = [pl.BlockSpec((bm, HP), lambda i: (i, 0))]
    if final:
        shapes += [jax.ShapeDtypeStruct((NP, H), F32)] * 2
        specs += [pl.BlockSpec((bm, H), lambda i: (i, 0))] * 2
    out = pl.pallas_call(
        functools.partial(_gru_body, final=final, bm=bm),
        grid=(NP // bm,),
        in_specs=[
            pl.BlockSpec((2, bm, HP), lambda i: (0, i, 0)),
            pl.BlockSpec((bm, HP), lambda i: (i, 0)),
            pl.BlockSpec((HP, 3 * 128), lambda i: (0, 0)),
            pl.BlockSpec((HP, 3 * 128), lambda i: (0, 0)),
            pl.BlockSpec((1, 3 * 128), lambda i: (0, 0)),
            pl.BlockSpec((1, H), lambda i: (0, 0)),
            pl.BlockSpec((1, HP), lambda i: (0, 0)),
            pl.BlockSpec((H, 1), lambda i: (0, 0)),
            pl.BlockSpec((1, 1), lambda i: (0, 0)),
        ],
        out_specs=specs[:nout],
        out_shape=shapes[:nout],
    )(parts, hid, wih, whh, bih, bhhn, bc, ww, bw)
    return out


def _readout_body(hw_hbm, hm_hbm, gid_hbm, zero_hbm, ninf_hbm,
                  sum_hbm, max_hbm, hw_v, hm_v, ids_v, sacc, macc):
    c = lax.axis_index("c")
    s = lax.axis_index("s")
    wid = s * 2 + c
    pltpu.sync_copy(hw_hbm.at[pl.ds(wid * NPW, NPW)], hw_v)
    pltpu.sync_copy(hm_hbm.at[pl.ds(wid * NPW, NPW)], hm_v)
    pltpu.sync_copy(gid_hbm.at[wid], ids_v.at[pl.ds(0, NPW)])
    pltpu.sync_copy(zero_hbm, sacc)
    pltpu.sync_copy(ninf_hbm, macc)

    def body(n, carry):
        gid = ids_v[pl.ds(n, 16)][0]
        for ci in range(H // 16):
            sl = pl.ds(ci * 16, 16)
            sacc[gid, sl] = sacc[gid, sl] + hw_v[n, sl]
            macc[gid, sl] = jnp.maximum(macc[gid, sl], hm_v[n, sl])
        return carry

    lax.fori_loop(0, NPW, body, 0)
    pltpu.sync_copy(sacc, sum_hbm.at[wid])
    pltpu.sync_copy(macc, max_hbm.at[wid])


def _sc_readout(hw, hm, gids, zero_g, ninf_g):
    mesh = plsc.VectorSubcoreMesh(core_axis_name="c", subcore_axis_name="s")
    return pl.kernel(
        _readout_body,
        out_type=(jax.ShapeDtypeStruct((NW, G, H), F32),
                  jax.ShapeDtypeStruct((NW, G, H), F32)),
        mesh=mesh,
        compiler_params=pltpu.CompilerParams(use_tc_tiling_on_sc=False),
        scratch_types=[
            pltpu.VMEM((NPW, H), F32),
            pltpu.VMEM((NPW, H), F32),
            pltpu.VMEM((NPW + 16,), jnp.int32),
            pltpu.VMEM((G, H), F32),
            pltpu.VMEM((G, H), F32),
        ],
    )(hw, hm, gids, zero_g, ninf_g)


def _final_body(s_ref, m_ref, wa_ref, wb_ref, b_ref, o_ref):
    hs = s_ref[0]
    hm = m_ref[0]
    for i in range(1, NW):
        hs = hs + s_ref[i]
        hm = jnp.maximum(hm, m_ref[i])
    o_ref[...] = (jnp.dot(hs.astype(BF16), wa_ref[...],
                          preferred_element_type=F32)
                  + jnp.dot(hm.astype(BF16), wb_ref[...],
                            preferred_element_type=F32)
                  + b_ref[...])


def _final(sum_parts, max_parts, wffA, wffB, bff):
    return pl.pallas_call(
        _final_body,
        in_specs=[
            pl.BlockSpec((NW, G, H), lambda: (0, 0, 0)),
            pl.BlockSpec((NW, G, H), lambda: (0, 0, 0)),
            pl.BlockSpec((H, C), lambda: (0, 0)),
            pl.BlockSpec((H, C), lambda: (0, 0)),
            pl.BlockSpec((1, C), lambda: (0, 0)),
        ],
        out_specs=pl.BlockSpec((G, C), lambda: (0, 0)),
        out_shape=jax.ShapeDtypeStruct((G, C), F32),
    )(sum_parts, max_parts, wffA, wffB, bff)


def kernel(node_feats, edge_feats, edge_index, graph_ids, W_proj, b_proj,
           W_e1, b_e1, W_e2, b_e2, b_conv, W_ih, W_hh, b_ih, b_hh,
           W_w, b_w, W_ff, b_ff):
    EA = NW * 3 * 128
    src_f = jnp.pad(edge_index[0], (0, EP - E))
    dst_f = jnp.pad(edge_index[1], (0, EP - E))
    srcA = src_f[:EA].reshape(NW, 3, 128)
    srcB = src_f[EA:].reshape(NW, 2, 128)
    dstA = dst_f[:EA].reshape(NW, 3, 128)
    dstB = dst_f[EA:].reshape(NW, 2, 128)
    gids = jnp.pad(graph_ids, (0, NP - N)).reshape(NW, NPW)

    wproj = jnp.pad(W_proj.astype(BF16), ((0, 0), (0, HP - H)))
    bproj = jnp.pad(b_proj.reshape(1, H), ((0, 0), (0, HP - H)))
    we1 = W_e1.astype(BF16)
    w2cat = jnp.pad(
        W_e2.reshape(EH, H, H).transpose(1, 0, 2).reshape(H, EH * H)
        .astype(BF16), ((0, HP - H), (0, 0)))
    smat = jnp.repeat(jnp.eye(EH, dtype=BF16), H, axis=1)
    b2 = jnp.pad(b_e2.reshape(H, H).astype(BF16), ((0, HP - H), (0, 0)))
    def _gatepad(wT):
        w3 = wT.astype(BF16).reshape(H, 3, H)
        return jnp.pad(w3, ((0, HP - H), (0, 0), (0, 128 - H))).reshape(
            HP, 3 * 128)
    wih = _gatepad(W_ih.T)
    whh = _gatepad(W_hh.T)
    bih = jnp.pad((b_ih + b_hh).reshape(3, H),
                  ((0, 0), (0, 128 - H))).reshape(1, 3 * 128)
    bhhn = b_hh[2 * H:].reshape(1, H)
    bc = jnp.zeros((1, HP), F32).at[:, :H].set(b_conv.reshape(1, H))
    ww16 = W_w.astype(BF16)
    bw = b_w.reshape(1, 1)
    wffA = W_ff[:H].astype(BF16)
    wffB = W_ff[H:].astype(BF16)
    bff = b_ff.reshape(1, C)

    zeros_tile = jnp.zeros((128, HP), F32)
    zeros_g = jnp.zeros((G, H), F32)
    ninf_g = jnp.full((G, H), NEG_INF, F32)

    h = _proj(node_feats, wproj, bproj, nvalid=N, mout=NP)
    z = _proj(edge_feats, we1, b_e1.reshape(1, EH), nvalid=E, mout=EP)

    zA = z[:EA]
    zB = z[EA:]
    hidden = h
    for step in range(STEPS):
        gA = _sc_gather(hidden, srcA, 3)
        mA = _messages(gA, zA, w2cat, smat, b2, evalid=None)
        gB = _sc_gather(hidden, srcB, 2)
        mB = _messages(gB, zB, w2cat, smat, b2, evalid=E - EA)
        parts = _sc_scatter(mA, mB, dstA, dstB, zeros_tile, 3, 2)
        final = step == STEPS - 1
        out = _gru(parts, hidden, wih, whh, bih, bhhn, bc,
                   ww16, bw, final=final)
        if final:
            hidden, hw, hm = out
        else:
            (hidden,) = out

    sum_parts, max_parts = _sc_readout(hw, hm, gids, zeros_g, ninf_g)
    return _final(sum_parts, max_parts, wffA, wffB, bff)


# speedup vs baseline: 1.0307x; 1.0307x over previous
import functools

import jax
import jax.numpy as jnp
from jax import lax
from jax.experimental import pallas as pl
from jax.experimental.pallas import tpu as pltpu
from jax.experimental.pallas import tpu_sc as plsc

N = 10000
E = 20000
G = 256
H = 64
EH = 32
STEPS = 3
C = 100

NP = 10240
EP = 20480
NW = 32
EPW = EP // NW
NCH = EPW // 128
NPW = NP // NW
NPT = NP // 16
HP = 128

F32 = jnp.float32
BF16 = jnp.bfloat16
NEG_INF = float("-inf")


def _proj_body(x_ref, w_ref, b_ref, o_ref, *, nvalid, bm):
    i = pl.program_id(0)
    y = jnp.dot(x_ref[...].astype(BF16), w_ref[...],
                preferred_element_type=F32) + b_ref[...]
    y = jnp.maximum(y, 0.0)
    rows = i * bm + lax.broadcasted_iota(jnp.int32, (bm, 1), 0)
    o_ref[...] = jnp.where(rows < nvalid, y, 0.0)


def _proj(x, w, b, nvalid, mout, bm=2048):
    k = x.shape[1]
    n = w.shape[1]
    return pl.pallas_call(
        functools.partial(_proj_body, nvalid=nvalid, bm=bm),
        grid=(mout // bm,),
        in_specs=[
            pl.BlockSpec((bm, k), lambda i: (i, 0)),
            pl.BlockSpec((k, n), lambda i: (0, 0)),
            pl.BlockSpec((1, n), lambda i: (0, 0)),
        ],
        out_specs=pl.BlockSpec((bm, n), lambda i: (i, 0)),
        out_shape=jax.ShapeDtypeStruct((mout, n), F32),
    )(x, w, b)


def _gather_body(tbl_hbm, idx_hbm, out_hbm, idx_v, rows_v, sem, *, nch):
    c = lax.axis_index("c")
    s = lax.axis_index("s")
    wid = s * 2 + c
    epw = nch * 128
    pltpu.sync_copy(idx_hbm.at[wid], idx_v)
    copies = []
    for j in range(nch):
        copies.append(
            pltpu.async_copy(tbl_hbm.at[idx_v.at[j]],
                             rows_v.at[pl.ds(j * 128, 128)], sem))
    for cp in copies:
        cp.wait()
    pltpu.sync_copy(rows_v, out_hbm.at[pl.ds(wid * epw, epw)])


def _sc_gather(table, idx, nch):
    mesh = plsc.VectorSubcoreMesh(core_axis_name="c", subcore_axis_name="s")
    return pl.kernel(
        functools.partial(_gather_body, nch=nch),
        out_type=jax.ShapeDtypeStruct((NW * nch * 128, HP), F32),
        mesh=mesh,
        scratch_types=[
            pltpu.VMEM((nch, 128), jnp.int32),
            pltpu.VMEM((nch * 128, HP), F32),
            pltpu.SemaphoreType.DMA,
        ],
    )(table, idx)


def _msg_body(g_ref, z_ref, w2_ref, s_ref, b2_ref, o_ref, *, bm, evalid):
    i = pl.program_id(0)
    gb = g_ref[...].astype(BF16)
    zb = z_ref[...].astype(BF16)
    ck = 256
    acc256 = jnp.zeros((bm, ck), F32)
    for c in range(EH * H // ck):
        sl = pl.ds(c * ck, ck)
        yc = jnp.dot(gb, w2_ref[:, sl], preferred_element_type=F32)
        zc = jnp.dot(zb, s_ref[:, sl], preferred_element_type=F32)
        acc256 = acc256 + yc * zc
    acc128 = acc256[:, :128] + acc256[:, 128:]
    m = acc128[:, :H] + acc128[:, H:] + jnp.dot(gb, b2_ref[...],
                                                preferred_element_type=F32)
    m128 = jnp.concatenate([m, jnp.zeros((bm, HP - H), F32)], axis=1)
    if evalid is not None:
        rows = i * bm + lax.broadcasted_iota(jnp.int32, (bm, 1), 0)
        m128 = jnp.where(rows < evalid, m128, 0.0)
    o_ref[...] = m128


def _messages(g, z, w2cat, smat, b2, evalid, zoff, bm=512):
    ep = g.shape[0]
    return pl.pallas_call(
        functools.partial(_msg_body, bm=bm, evalid=evalid),
        grid=(ep // bm,),
        in_specs=[
            pl.BlockSpec((bm, HP), lambda i: (i, 0)),
            pl.BlockSpec((bm, EH), lambda i: (i + zoff, 0)),
            pl.BlockSpec((HP, EH * H), lambda i: (0, 0)),
            pl.BlockSpec((EH, EH * H), lambda i: (0, 0)),
            pl.BlockSpec((HP, H), lambda i: (0, 0)),
        ],
        out_specs=pl.BlockSpec((bm, HP), lambda i: (i, 0)),
        out_shape=jax.ShapeDtypeStruct((ep, HP), F32),
    )(g, z, w2cat, smat, b2)


def _scatter_body(ma_hbm, mb_hbm, idxa_hbm, idxb_hbm, zero_hbm, part_hbm,
                  m_v0, m_v1, idxa_v, idxb_v, acc_sh, sem0, sem1, *,
                  nch_a, nch_b):
    c = lax.axis_index("c")
    s = lax.axis_index("s")
    wid = s * 2 + c
    bufs = (m_v0, m_v1)
    sems = (sem0, sem1)
    def chunk_src(j):
        if j < nch_a:
            return ma_hbm.at[pl.ds(wid * nch_a * 128 + j * 128, 128)]
        return mb_hbm.at[pl.ds(wid * nch_b * 128 + (j - nch_a) * 128, 128)]
    def chunk_idx(j):
        if j < nch_a:
            return idxa_v.at[j]
        return idxb_v.at[j - nch_a]
    nch = nch_a + nch_b
    pltpu.sync_copy(zero_hbm, m_v0)
    for j in range(NPT // 128):
        pltpu.sync_copy(m_v0, acc_sh.at[pl.ds(s * NPT + j * 128, 128)])
    pltpu.sync_copy(idxa_hbm.at[wid], idxa_v)
    pltpu.sync_copy(idxb_hbm.at[wid], idxb_v)
    cps = [None, None]
    cps[0] = pltpu.async_copy(chunk_src(0), m_v0, sem0)
    plsc.subcore_barrier()
    for j in range(nch):
        if j + 1 < nch:
            cps[(j + 1) % 2] = pltpu.async_copy(
                chunk_src(j + 1), bufs[(j + 1) % 2], sems[(j + 1) % 2])
        cps[j % 2].wait()
        pltpu.sync_copy(bufs[j % 2], acc_sh.at[chunk_idx(j)], add=True)
    plsc.subcore_barrier()
    pltpu.sync_copy(acc_sh.at[pl.ds(s * NPT, NPT)],
                    part_hbm.at[c, pl.ds(s * NPT, NPT)])


def _sc_scatter(ma, mb, idxa, idxb, zeros_tile, nch_a, nch_b):
    mesh = plsc.VectorSubcoreMesh(core_axis_name="c", subcore_axis_name="s")
    return pl.kernel(
        functools.partial(_scatter_body, nch_a=nch_a, nch_b=nch_b),
        out_type=jax.ShapeDtypeStruct((2, NP, HP), F32),
        mesh=mesh,
        scratch_types=[
            pltpu.VMEM((128, HP), F32),
            pltpu.VMEM((128, HP), F32),
            pltpu.VMEM((nch_a, 128), jnp.int32),
            pltpu.VMEM((nch_b, 128), jnp.int32),
            pltpu.VMEM_SHARED((NP, HP), F32),
            pltpu.SemaphoreType.DMA,
            pltpu.SemaphoreType.DMA,
        ],
    )(ma, mb, idxa, idxb, zeros_tile)


def _gru_body(p_ref, hid_ref, wih_ref, whh_ref, bih_ref, bhh_ref, bc_ref,
              ww_ref, bw_ref, h_ref, *extra_refs, final, bm):
    i = pl.program_id(0)
    hid = hid_ref[:, :H]
    x = jnp.maximum(p_ref[0] + p_ref[1] + bc_ref[...], 0.0)
    gs = (jnp.dot(x.astype(BF16), wih_ref[...], preferred_element_type=F32)
          + jnp.dot(hid_ref[...].astype(BF16), whh_ref[...],
                    preferred_element_type=F32) + bih_ref[...])
    gh_n = (jnp.dot(hid_ref[...].astype(BF16), whh_ref[:, 2 * 128:2 * 128 + H],
                    preferred_element_type=F32) + bhh_ref[...])
    r = jax.nn.sigmoid(gs[:, 0:H])
    zg = jax.nn.sigmoid(gs[:, 128:128 + H])
    n = jnp.tanh(gs[:, 256:256 + H] - gh_n + r * gh_n)
    hn = (1.0 - zg) * n + zg * hid
    rows = i * bm + lax.broadcasted_iota(jnp.int32, (bm, 1), 0)
    valid = rows < N
    hn = jnp.where(valid, hn, 0.0)
    h_ref[...] = jnp.concatenate([hn, jnp.zeros((bm, HP - H), F32)], axis=1)
    if final:
        hw_ref, hm_ref = extra_refs
        wv = jax.nn.sigmoid(jnp.dot(hn.astype(BF16), ww_ref[...],
                                    preferred_element_type=F32) + bw_ref[...])
        hw_ref[...] = jnp.where(valid, hn * wv, 0.0)
        hm_ref[...] = jnp.where(valid, hn, NEG_INF)


def _gru(parts, hid, wih, whh, bih, bhhn, bc, ww, bw, final, bm=1024):
    nout = 3 if final else 1
    shapes = [jax.ShapeDtypeStruct((NP, HP), F32)]
    specs = [pl.BlockSpec((bm, HP), lambda i: (i, 0))]
    if final:
        shapes += [jax.ShapeDtypeStruct((NP, H), F32)] * 2
        specs += [pl.BlockSpec((bm, H), lambda i: (i, 0))] * 2
    out = pl.pallas_call(
        functools.partial(_gru_body, final=final, bm=bm),
        grid=(NP // bm,),
        in_specs=[
            pl.BlockSpec((2, bm, HP), lambda i: (0, i, 0)),
            pl.BlockSpec((bm, HP), lambda i: (i, 0)),
            pl.BlockSpec((HP, 3 * 128), lambda i: (0, 0)),
            pl.BlockSpec((HP, 3 * 128), lambda i: (0, 0)),
            pl.BlockSpec((1, 3 * 128), lambda i: (0, 0)),
            pl.BlockSpec((1, H), lambda i: (0, 0)),
            pl.BlockSpec((1, HP), lambda i: (0, 0)),
            pl.BlockSpec((H, 1), lambda i: (0, 0)),
            pl.BlockSpec((1, 1), lambda i: (0, 0)),
        ],
        out_specs=specs[:nout],
        out_shape=shapes[:nout],
    )(parts, hid, wih, whh, bih, bhhn, bc, ww, bw)
    return out


def _readout_body(hw_hbm, hm_hbm, gid_hbm, zero_hbm, ninf_hbm,
                  sum_hbm, max_hbm, hw_v, hm_v, ids_v, sacc, macc):
    c = lax.axis_index("c")
    s = lax.axis_index("s")
    wid = s * 2 + c
    pltpu.sync_copy(hw_hbm.at[pl.ds(wid * NPW, NPW)], hw_v)
    pltpu.sync_copy(hm_hbm.at[pl.ds(wid * NPW, NPW)], hm_v)
    pltpu.sync_copy(gid_hbm.at[wid], ids_v.at[pl.ds(0, NPW)])
    pltpu.sync_copy(zero_hbm, sacc)
    pltpu.sync_copy(ninf_hbm, macc)

    def body(n, carry):
        gid = ids_v[pl.ds(n, 16)][0]
        for ci in range(H // 16):
            sl = pl.ds(ci * 16, 16)
            sacc[gid, sl] = sacc[gid, sl] + hw_v[n, sl]
            macc[gid, sl] = jnp.maximum(macc[gid, sl], hm_v[n, sl])
        return carry

    lax.fori_loop(0, NPW, body, 0)
    pltpu.sync_copy(sacc, sum_hbm.at[wid])
    pltpu.sync_copy(macc, max_hbm.at[wid])


def _sc_readout(hw, hm, gids, zero_g, ninf_g):
    mesh = plsc.VectorSubcoreMesh(core_axis_name="c", subcore_axis_name="s")
    return pl.kernel(
        _readout_body,
        out_type=(jax.ShapeDtypeStruct((NW, G, H), F32),
                  jax.ShapeDtypeStruct((NW, G, H), F32)),
        mesh=mesh,
        compiler_params=pltpu.CompilerParams(use_tc_tiling_on_sc=False),
        scratch_types=[
            pltpu.VMEM((NPW, H), F32),
            pltpu.VMEM((NPW, H), F32),
            pltpu.VMEM((NPW + 16,), jnp.int32),
            pltpu.VMEM((G, H), F32),
            pltpu.VMEM((G, H), F32),
        ],
    )(hw, hm, gids, zero_g, ninf_g)


def _final_body(s_ref, m_ref, wa_ref, wb_ref, b_ref, o_ref):
    hs = s_ref[0]
    hm = m_ref[0]
    for i in range(1, NW):
        hs = hs + s_ref[i]
        hm = jnp.maximum(hm, m_ref[i])
    o_ref[...] = (jnp.dot(hs.astype(BF16), wa_ref[...],
                          preferred_element_type=F32)
                  + jnp.dot(hm.astype(BF16), wb_ref[...],
                            preferred_element_type=F32)
                  + b_ref[...])


def _final(sum_parts, max_parts, wffA, wffB, bff):
    return pl.pallas_call(
        _final_body,
        in_specs=[
            pl.BlockSpec((NW, G, H), lambda: (0, 0, 0)),
            pl.BlockSpec((NW, G, H), lambda: (0, 0, 0)),
            pl.BlockSpec((H, C), lambda: (0, 0)),
            pl.BlockSpec((H, C), lambda: (0, 0)),
            pl.BlockSpec((1, C), lambda: (0, 0)),
        ],
        out_specs=pl.BlockSpec((G, C), lambda: (0, 0)),
        out_shape=jax.ShapeDtypeStruct((G, C), F32),
    )(sum_parts, max_parts, wffA, wffB, bff)


def kernel(node_feats, edge_feats, edge_index, graph_ids, W_proj, b_proj,
           W_e1, b_e1, W_e2, b_e2, b_conv, W_ih, W_hh, b_ih, b_hh,
           W_w, b_w, W_ff, b_ff):
    EA = NW * 3 * 128
    src_f = jnp.pad(edge_index[0], (0, EP - E))
    dst_f = jnp.pad(edge_index[1], (0, EP - E))
    srcA = src_f[:EA].reshape(NW, 3, 128)
    srcB = src_f[EA:].reshape(NW, 2, 128)
    dstA = dst_f[:EA].reshape(NW, 3, 128)
    dstB = dst_f[EA:].reshape(NW, 2, 128)
    gids = jnp.pad(graph_ids, (0, NP - N)).reshape(NW, NPW)

    wproj = jnp.pad(W_proj.astype(BF16), ((0, 0), (0, HP - H)))
    bproj = jnp.pad(b_proj.reshape(1, H), ((0, 0), (0, HP - H)))
    we1 = W_e1.astype(BF16)
    w2cat = jnp.pad(
        W_e2.reshape(EH, H, H).transpose(1, 0, 2).reshape(H, EH * H)
        .astype(BF16), ((0, HP - H), (0, 0)))
    smat = jnp.repeat(jnp.eye(EH, dtype=BF16), H, axis=1)
    b2 = jnp.pad(b_e2.reshape(H, H).astype(BF16), ((0, HP - H), (0, 0)))
    def _gatepad(wT):
        w3 = wT.astype(BF16).reshape(H, 3, H)
        return jnp.pad(w3, ((0, HP - H), (0, 0), (0, 128 - H))).reshape(
            HP, 3 * 128)
    wih = _gatepad(W_ih.T)
    whh = _gatepad(W_hh.T)
    bih = jnp.pad((b_ih + b_hh).reshape(3, H),
                  ((0, 0), (0, 128 - H))).reshape(1, 3 * 128)
    bhhn = b_hh[2 * H:].reshape(1, H)
    bc = jnp.zeros((1, HP), F32).at[:, :H].set(b_conv.reshape(1, H))
    ww16 = W_w.astype(BF16)
    bw = b_w.reshape(1, 1)
    wffA = W_ff[:H].astype(BF16)
    wffB = W_ff[H:].astype(BF16)
    bff = b_ff.reshape(1, C)

    zeros_tile = jnp.zeros((128, HP), F32)
    zeros_g = jnp.zeros((G, H), F32)
    ninf_g = jnp.full((G, H), NEG_INF, F32)

    h = _proj(node_feats, wproj, bproj, nvalid=N, mout=NP)
    z = _proj(edge_feats, we1, b_e1.reshape(1, EH), nvalid=E, mout=EP)

    hidden = h
    for step in range(STEPS):
        gA = _sc_gather(hidden, srcA, 3)
        mA = _messages(gA, z, w2cat, smat, b2, evalid=None, zoff=0)
        gB = _sc_gather(hidden, srcB, 2)
        mB = _messages(gB, z, w2cat, smat, b2, evalid=E - EA, zoff=EA // 512)
        parts = _sc_scatter(mA, mB, dstA, dstB, zeros_tile, 3, 2)
        final = step == STEPS - 1
        out = _gru(parts, hidden, wih, whh, bih, bhhn, bc,
                   ww16, bw, final=final)
        if final:
            hidden, hw, hm = out
        else:
            (hidden,) = out

    sum_parts, max_parts = _sc_readout(hw, hm, gids, zeros_g, ninf_g)
    return _final(sum_parts, max_parts, wffA, wffB, bff)


# speedup vs baseline: 1.0323x; 1.0015x over previous
import functools

import jax
import jax.numpy as jnp
from jax import lax
from jax.experimental import pallas as pl
from jax.experimental.pallas import tpu as pltpu
from jax.experimental.pallas import tpu_sc as plsc

N = 10000
E = 20000
G = 256
H = 64
EH = 32
STEPS = 3
C = 100

NP = 10240
EP = 20480
NW = 32
EPW = EP // NW
NCH = EPW // 128
NPW = NP // NW
NPT = NP // 16
HP = 128

F32 = jnp.float32
BF16 = jnp.bfloat16
NEG_INF = float("-inf")


def _proj_body(x_ref, w_ref, b_ref, o_ref, *, nvalid, bm):
    i = pl.program_id(0)
    y = jnp.dot(x_ref[...].astype(BF16), w_ref[...],
                preferred_element_type=F32) + b_ref[...]
    y = jnp.maximum(y, 0.0)
    rows = i * bm + lax.broadcasted_iota(jnp.int32, (bm, 1), 0)
    o_ref[...] = jnp.where(rows < nvalid, y, 0.0)


def _proj(x, w, b, nvalid, mout, bm=2048):
    k = x.shape[1]
    n = w.shape[1]
    return pl.pallas_call(
        functools.partial(_proj_body, nvalid=nvalid, bm=bm),
        grid=(mout // bm,),
        in_specs=[
            pl.BlockSpec((bm, k), lambda i: (i, 0)),
            pl.BlockSpec((k, n), lambda i: (0, 0)),
            pl.BlockSpec((1, n), lambda i: (0, 0)),
        ],
        out_specs=pl.BlockSpec((bm, n), lambda i: (i, 0)),
        out_shape=jax.ShapeDtypeStruct((mout, n), F32),
    )(x, w, b)


def _gather_body(tbl_hbm, idx_hbm, out_hbm, idx_v, rows_v, sem, *, nch):
    c = lax.axis_index("c")
    s = lax.axis_index("s")
    wid = s * 2 + c
    epw = nch * 128
    pltpu.sync_copy(idx_hbm.at[wid], idx_v)
    copies = []
    for j in range(nch):
        copies.append(
            pltpu.async_copy(tbl_hbm.at[idx_v.at[j]],
                             rows_v.at[pl.ds(j * 128, 128)], sem))
    for cp in copies:
        cp.wait()
    pltpu.sync_copy(rows_v, out_hbm.at[pl.ds(wid * epw, epw)])


def _sc_gather(table, idx, nch):
    mesh = plsc.VectorSubcoreMesh(core_axis_name="c", subcore_axis_name="s")
    return pl.kernel(
        functools.partial(_gather_body, nch=nch),
        out_type=jax.ShapeDtypeStruct((NW * nch * 128, HP), F32),
        mesh=mesh,
        scratch_types=[
            pltpu.VMEM((nch, 128), jnp.int32),
            pltpu.VMEM((nch * 128, HP), F32),
            pltpu.SemaphoreType.DMA,
        ],
    )(table, idx)


def _msg_body(g_ref, z_ref, w2_ref, s_ref, b2_ref, o_ref, *, bm, evalid):
    i = pl.program_id(0)
    gb = g_ref[...].astype(BF16)
    zb = z_ref[...].astype(BF16)
    ck = 256
    acc256 = jnp.zeros((bm, ck), F32)
    for c in range(EH * H // ck):
        sl = pl.ds(c * ck, ck)
        yc = jnp.dot(gb, w2_ref[:, sl], preferred_element_type=F32)
        zc = jnp.dot(zb, s_ref[:, sl], preferred_element_type=F32)
        acc256 = acc256 + yc * zc
    acc128 = acc256[:, :128] + acc256[:, 128:]
    m = acc128[:, :H] + acc128[:, H:] + jnp.dot(gb, b2_ref[...],
                                                preferred_element_type=F32)
    m128 = jnp.concatenate([m, jnp.zeros((bm, HP - H), F32)], axis=1)
    if evalid is not None:
        rows = i * bm + lax.broadcasted_iota(jnp.int32, (bm, 1), 0)
        m128 = jnp.where(rows < evalid, m128, 0.0)
    o_ref[...] = m128


def _messages(g, z, w2cat, smat, b2, evalid, zoff, bm=512):
    ep = g.shape[0]
    return pl.pallas_call(
        functools.partial(_msg_body, bm=bm, evalid=evalid),
        grid=(ep // bm,),
        in_specs=[
            pl.BlockSpec((bm, HP), lambda i: (i, 0)),
            pl.BlockSpec((bm, EH), lambda i: (i + zoff, 0)),
            pl.BlockSpec((HP, EH * H), lambda i: (0, 0)),
            pl.BlockSpec((EH, EH * H), lambda i: (0, 0)),
            pl.BlockSpec((HP, H), lambda i: (0, 0)),
        ],
        out_specs=pl.BlockSpec((bm, HP), lambda i: (i, 0)),
        out_shape=jax.ShapeDtypeStruct((ep, HP), F32),
    )(g, z, w2cat, smat, b2)


def _scatter_body(ma_hbm, mb_hbm, idxa_hbm, idxb_hbm, zero_hbm, part_hbm,
                  m_v0, m_v1, idxa_v, idxb_v, acc_sh, sem0, sem1, *,
                  nch_a, nch_b):
    c = lax.axis_index("c")
    s = lax.axis_index("s")
    wid = s * 2 + c
    bufs = (m_v0, m_v1)
    sems = (sem0, sem1)
    def chunk_src(j):
        if j < nch_a:
            return ma_hbm.at[pl.ds(wid * nch_a * 128 + j * 128, 128)]
        return mb_hbm.at[pl.ds(wid * nch_b * 128 + (j - nch_a) * 128, 128)]
    def chunk_idx(j):
        if j < nch_a:
            return idxa_v.at[j]
        return idxb_v.at[j - nch_a]
    nch = nch_a + nch_b
    pltpu.sync_copy(zero_hbm, m_v0)
    for j in range(NPT // 128):
        pltpu.sync_copy(m_v0, acc_sh.at[pl.ds(s * NPT + j * 128, 128)])
    pltpu.sync_copy(idxa_hbm.at[wid], idxa_v)
    pltpu.sync_copy(idxb_hbm.at[wid], idxb_v)
    cps = [None, None]
    cps[0] = pltpu.async_copy(chunk_src(0), m_v0, sem0)
    plsc.subcore_barrier()
    for j in range(nch):
        if j + 1 < nch:
            cps[(j + 1) % 2] = pltpu.async_copy(
                chunk_src(j + 1), bufs[(j + 1) % 2], sems[(j + 1) % 2])
        cps[j % 2].wait()
        pltpu.sync_copy(bufs[j % 2], acc_sh.at[chunk_idx(j)], add=True)
    plsc.subcore_barrier()
    pltpu.sync_copy(acc_sh.at[pl.ds(s * NPT, NPT)],
                    part_hbm.at[c, pl.ds(s * NPT, NPT)])


def _sc_scatter(ma, mb, idxa, idxb, zeros_tile, nch_a, nch_b):
    mesh = plsc.VectorSubcoreMesh(core_axis_name="c", subcore_axis_name="s")
    return pl.kernel(
        functools.partial(_scatter_body, nch_a=nch_a, nch_b=nch_b),
        out_type=jax.ShapeDtypeStruct((2, NP, HP), F32),
        mesh=mesh,
        scratch_types=[
            pltpu.VMEM((128, HP), F32),
            pltpu.VMEM((128, HP), F32),
            pltpu.VMEM((nch_a, 128), jnp.int32),
            pltpu.VMEM((nch_b, 128), jnp.int32),
            pltpu.VMEM_SHARED((NP, HP), F32),
            pltpu.SemaphoreType.DMA,
            pltpu.SemaphoreType.DMA,
        ],
    )(ma, mb, idxa, idxb, zeros_tile)


def _gru_body(p_ref, hid_ref, wih_ref, whh_ref, bih_ref, bhh_ref, bc_ref,
              ww_ref, bw_ref, h_ref, *extra_refs, final, bm):
    i = pl.program_id(0)
    hid = hid_ref[:, :H]
    x = jnp.maximum(p_ref[0] + p_ref[1] + bc_ref[...], 0.0)
    gs = (jnp.dot(x.astype(BF16), wih_ref[...], preferred_element_type=F32)
          + jnp.dot(hid_ref[...].astype(BF16), whh_ref[...],
                    preferred_element_type=F32) + bih_ref[...])
    gh_n = (jnp.dot(hid_ref[...].astype(BF16), whh_ref[:, 2 * 128:2 * 128 + H],
                    preferred_element_type=F32) + bhh_ref[...])
    r = jax.nn.sigmoid(gs[:, 0:H])
    zg = jax.nn.sigmoid(gs[:, 128:128 + H])
    n = jnp.tanh(gs[:, 256:256 + H] - gh_n + r * gh_n)
    hn = (1.0 - zg) * n + zg * hid
    rows = i * bm + lax.broadcasted_iota(jnp.int32, (bm, 1), 0)
    valid = rows < N
    hn = jnp.where(valid, hn, 0.0)
    h_ref[...] = jnp.concatenate([hn, jnp.zeros((bm, HP - H), F32)], axis=1)
    if final:
        hw_ref, hm_ref = extra_refs
        wv = jax.nn.sigmoid(jnp.dot(hn.astype(BF16), ww_ref[...],
                                    preferred_element_type=F32) + bw_ref[...])
        hw_ref[...] = jnp.where(valid, hn * wv, 0.0)
        hm_ref[...] = jnp.where(valid, hn, NEG_INF)


def _gru(parts, hid, wih, whh, bih, bhhn, bc, ww, bw, final, bm=1024):
    nout = 3 if final else 1
    shapes = [jax.ShapeDtypeStruct((NP, HP), F32)]
    specs = [pl.BlockSpec((bm, HP), lambda i: (i, 0))]
    if final:
        shapes += [jax.ShapeDtypeStruct((NP, H), F32)] * 2
        specs += [pl.BlockSpec((bm, H), lambda i: (i, 0))] * 2
    out = pl.pallas_call(
        functools.partial(_gru_body, final=final, bm=bm),
        grid=(NP // bm,),
        in_specs=[
            pl.BlockSpec((2, bm, HP), lambda i: (0, i, 0)),
            pl.BlockSpec((bm, HP), lambda i: (i, 0)),
            pl.BlockSpec((HP, 3 * 128), lambda i: (0, 0)),
            pl.BlockSpec((HP, 3 * 128), lambda i: (0, 0)),
            pl.BlockSpec((1, 3 * 128), lambda i: (0, 0)),
            pl.BlockSpec((1, H), lambda i: (0, 0)),
            pl.BlockSpec((1, HP), lambda i: (0, 0)),
            pl.BlockSpec((H, 1), lambda i: (0, 0)),
            pl.BlockSpec((1, 1), lambda i: (0, 0)),
        ],
        out_specs=specs[:nout],
        out_shape=shapes[:nout],
    )(parts, hid, wih, whh, bih, bhhn, bc, ww, bw)
    return out


def _readout_body(hw_hbm, hm_hbm, gid_hbm, zero_hbm, ninf_hbm,
                  sum_hbm, max_hbm, hw_v, hm_v, ids_v, sacc, macc):
    c = lax.axis_index("c")
    s = lax.axis_index("s")
    wid = s * 2 + c
    pltpu.sync_copy(hw_hbm.at[pl.ds(wid * NPW, NPW)], hw_v)
    pltpu.sync_copy(hm_hbm.at[pl.ds(wid * NPW, NPW)], hm_v)
    pltpu.sync_copy(gid_hbm.at[wid], ids_v.at[pl.ds(0, NPW)])
    pltpu.sync_copy(zero_hbm, sacc)
    pltpu.sync_copy(ninf_hbm, macc)

    def body(n, carry):
        gid = ids_v[pl.ds(n, 16)][0]
        for ci in range(H // 16):
            sl = pl.ds(ci * 16, 16)
            sacc[gid, sl] = sacc[gid, sl] + hw_v[n, sl]
            macc[gid, sl] = jnp.maximum(macc[gid, sl], hm_v[n, sl])
        return carry

    lax.fori_loop(0, NPW, body, 0)
    pltpu.sync_copy(sacc, sum_hbm.at[wid])
    pltpu.sync_copy(macc, max_hbm.at[wid])


def _sc_readout(hw, hm, gids, zero_g, ninf_g):
    mesh = plsc.VectorSubcoreMesh(core_axis_name="c", subcore_axis_name="s")
    return pl.kernel(
        _readout_body,
        out_type=(jax.ShapeDtypeStruct((NW, G, H), F32),
                  jax.ShapeDtypeStruct((NW, G, H), F32)),
        mesh=mesh,
        compiler_params=pltpu.CompilerParams(use_tc_tiling_on_sc=False),
        scratch_types=[
            pltpu.VMEM((NPW, H), F32),
            pltpu.VMEM((NPW, H), F32),
            pltpu.VMEM((NPW + 16,), jnp.int32),
            pltpu.VMEM((G, H), F32),
            pltpu.VMEM((G, H), F32),
        ],
    )(hw, hm, gids, zero_g, ninf_g)


def _final_body(s_ref, m_ref, wa_ref, wb_ref, b_ref, o_ref):
    hs = s_ref[0]
    hm = m_ref[0]
    for i in range(1, NW):
        hs = hs + s_ref[i]
        hm = jnp.maximum(hm, m_ref[i])
    o_ref[...] = (jnp.dot(hs.astype(BF16), wa_ref[...],
                          preferred_element_type=F32)
                  + jnp.dot(hm.astype(BF16), wb_ref[...],
                            preferred_element_type=F32)
                  + b_ref[...])


def _final(sum_parts, max_parts, wffA, wffB, bff):
    return pl.pallas_call(
        _final_body,
        in_specs=[
            pl.BlockSpec((NW, G, H), lambda: (0, 0, 0)),
            pl.BlockSpec((NW, G, H), lambda: (0, 0, 0)),
            pl.BlockSpec((H, C), lambda: (0, 0)),
            pl.BlockSpec((H, C), lambda: (0, 0)),
            pl.BlockSpec((1, C), lambda: (0, 0)),
        ],
        out_specs=pl.BlockSpec((G, C), lambda: (0, 0)),
        out_shape=jax.ShapeDtypeStruct((G, C), F32),
    )(sum_parts, max_parts, wffA, wffB, bff)


def kernel(node_feats, edge_feats, edge_index, graph_ids, W_proj, b_proj,
           W_e1, b_e1, W_e2, b_e2, b_conv, W_ih, W_hh, b_ih, b_hh,
           W_w, b_w, W_ff, b_ff):
    EA = NW * 3 * 128
    src_f = jnp.pad(edge_index[0], (0, EP - E))
    dst_f = jnp.pad(edge_index[1], (0, EP - E))
    srcA = src_f[:EA].reshape(NW, 3, 128)
    srcB = src_f[EA:].reshape(NW, 2, 128)
    dstA = dst_f[:EA].reshape(NW, 3, 128)
    dstB = dst_f[EA:].reshape(NW, 2, 128)
    gids = jnp.pad(graph_ids, (0, NP - N)).reshape(NW, NPW)

    wproj = jnp.pad(W_proj.astype(BF16), ((0, 0), (0, HP - H)))
    bproj = jnp.pad(b_proj.reshape(1, H), ((0, 0), (0, HP - H)))
    we1 = W_e1.astype(BF16)
    w2cat = jnp.pad(
        W_e2.reshape(EH, H, H).transpose(1, 0, 2).reshape(H, EH * H)
        .astype(BF16), ((0, HP - H), (0, 0)))
    smat = jnp.repeat(jnp.eye(EH, dtype=BF16), H, axis=1)
    b2 = jnp.pad(b_e2.reshape(H, H).astype(BF16), ((0, HP - H), (0, 0)))
    def _gatepad(wT):
        w3 = wT.astype(BF16).reshape(H, 3, H)
        return jnp.pad(w3, ((0, HP - H), (0, 0), (0, 128 - H))).reshape(
            HP, 3 * 128)
    wih = _gatepad(W_ih.T)
    whh = _gatepad(W_hh.T)
    bih = jnp.pad((b_ih + b_hh).reshape(3, H),
                  ((0, 0), (0, 128 - H))).reshape(1, 3 * 128)
    bhhn = b_hh[2 * H:].reshape(1, H)
    bc = jnp.zeros((1, HP), F32).at[:, :H].set(b_conv.reshape(1, H))
    ww16 = W_w.astype(BF16)
    bw = b_w.reshape(1, 1)
    wffA = W_ff[:H].astype(BF16)
    wffB = W_ff[H:].astype(BF16)
    bff = b_ff.reshape(1, C)

    zeros_tile = jnp.zeros((128, HP), F32)
    zeros_g = jnp.zeros((G, H), F32)
    ninf_g = jnp.full((G, H), NEG_INF, F32)

    h = _proj(node_feats, wproj, bproj, nvalid=N, mout=NP)
    z = _proj(edge_feats, we1, b_e1.reshape(1, EH), nvalid=E, mout=EP)

    hidden = h
    for step in range(STEPS):
        gA = _sc_gather(hidden, srcA, 3)
        gB = _sc_gather(hidden, srcB, 2)
        mA = _messages(gA, z, w2cat, smat, b2, evalid=None, zoff=0)
        mB = _messages(gB, z, w2cat, smat, b2, evalid=E - EA, zoff=EA // 512)
        parts = _sc_scatter(mA, mB, dstA, dstB, zeros_tile, 3, 2)
        final = step == STEPS - 1
        out = _gru(parts, hidden, wih, whh, bih, bhhn, bc,
                   ww16, bw, final=final)
        if final:
            hidden, hw, hm = out
        else:
            (hidden,) = out

    sum_parts, max_parts = _sc_readout(hw, hm, gids, zeros_g, ninf_g)
    return _final(sum_parts, max_parts, wffA, wffB, bff)
